# baseline (speedup 1.0000x reference)
import jax
import jax.numpy as jnp
from jax import lax
from jax.experimental import pallas as pl
from jax.experimental.pallas import tpu as pltpu
from jax.experimental.pallas import tpu_sc as plsc

NUM_FIELDS = 26
FIELD_DIM = 100000
BATCH = 16384
L = 16
NC, NS = 2, 16
NW = NC * NS
B_PER_W = BATCH // NW
N_PER_W = B_PER_W * NUM_FIELDS
CHUNK = 128
N_CHUNKS = N_PER_W // CHUNK
ROWS_PER_CHUNK = CHUNK // L
DIV26_MUL = 40330
DIV26_SHIFT = 20


def _body(x_hbm, tab_hbm, bias_hbm, out_hbm,
          idx_v, rows_v, didx_v, acc_sh, sem, sem2):
    cid = lax.axis_index("c")
    sid = lax.axis_index("s")
    wid = cid * NS + sid

    pltpu.sync_copy(x_hbm.at[pl.ds(wid * N_PER_W, N_PER_W)], idx_v)

    lane = lax.iota(jnp.int32, L)

    def add_offsets(k, carry):
        o = k * L
        idx_v[pl.ds(o, L)] = idx_v[pl.ds(o, L)] + ((lane + o) % NUM_FIELDS) * FIELD_DIM
        return carry

    lax.fori_loop(0, N_PER_W // L, add_offsets, 0)

    def fire(j, carry):
        o = j * CHUNK
        pltpu.make_async_copy(
            tab_hbm.at[0].at[idx_v.at[pl.ds(o, CHUNK)]],
            rows_v.at[pl.ds(o, CHUNK)],
            sem,
        ).start()
        return carry

    lax.fori_loop(0, N_CHUNKS, fire, 0)

    acc_base = sid * B_PER_W

    def build_didx(k, carry):
        o = k * L
        row = ((lane + o) * DIV26_MUL) >> DIV26_SHIFT
        didx_v[pl.ds(o, L)] = acc_base + row
        return carry

    lax.fori_loop(0, N_PER_W // L, build_didx, 0)

    pltpu.sync_copy(bias_hbm, acc_sh.at[pl.ds(acc_base, B_PER_W)])

    def drain(j, carry):
        o = j * CHUNK
        pltpu.make_async_copy(
            tab_hbm.at[0].at[idx_v.at[pl.ds(o, CHUNK)]],
            rows_v.at[pl.ds(o, CHUNK)],
            sem,
        ).wait()
        return carry

    lax.fori_loop(0, N_CHUNKS, drain, 0)

    def fire_scatter(j, carry):
        pltpu.async_copy(
            rows_v.at[pl.ds(j * CHUNK, CHUNK)],
            acc_sh.at[didx_v.at[pl.ds(j * CHUNK, CHUNK)]],
            sem2,
            add=True,
        )
        return carry

    lax.fori_loop(0, N_CHUNKS, fire_scatter, 0)

    def drain_scatter(j, carry):
        pltpu.make_async_copy(
            rows_v.at[pl.ds(j * CHUNK, CHUNK)],
            acc_sh.at[didx_v.at[pl.ds(j * CHUNK, CHUNK)]],
            sem2,
        ).wait()
        return carry

    lax.fori_loop(0, N_CHUNKS, drain_scatter, 0)

    pltpu.sync_copy(acc_sh.at[pl.ds(acc_base, B_PER_W)],
                    out_hbm.at[pl.ds(wid * B_PER_W, B_PER_W)])


@jax.jit
def _run(x_flat, table, bias_seed):
    mesh = plsc.VectorSubcoreMesh(core_axis_name="c", subcore_axis_name="s")
    return pl.kernel(
        _body,
        out_type=jax.ShapeDtypeStruct((BATCH,), jnp.float32),
        compiler_params=pltpu.CompilerParams(use_tc_tiling_on_sc=False),
        mesh=mesh,
        scratch_types=[
            pltpu.VMEM((N_PER_W,), jnp.int32),
            pltpu.VMEM((N_PER_W,), jnp.float32),
            pltpu.VMEM((N_PER_W,), jnp.int32),
            pltpu.VMEM_SHARED((NS * B_PER_W,), jnp.float32),
            pltpu.SemaphoreType.DMA,
            pltpu.SemaphoreType.DMA,
        ],
    )(x_flat, table, bias_seed)


def kernel(x, table, bias):
    x_flat = x.reshape(-1)
    tab_row = table.reshape(1, -1)
    bias_seed = jnp.broadcast_to(bias, (B_PER_W,))
    return _run(x_flat, tab_row, bias_seed).reshape(BATCH, 1)


# speedup vs baseline: 1.1200x; 1.1200x over previous
import jax
import jax.numpy as jnp
from jax import lax
from jax.experimental import pallas as pl
from jax.experimental.pallas import tpu as pltpu
from jax.experimental.pallas import tpu_sc as plsc

NUM_FIELDS = 26
FIELD_DIM = 100000
TOTAL_ROWS = NUM_FIELDS * FIELD_DIM
BATCH = 16384
L = 16
NC, NS = 2, 16
NW = NC * NS
B_PER_W = BATCH // NW
N_PER_W = B_PER_W * NUM_FIELDS
CHUNK = 128
N_CHUNKS = N_PER_W // CHUNK

RELAY_RB = 81248
RELAY_LAST = TOTAL_ROWS - 31 * RELAY_RB


def _gather_body(xt_hbm, flat_hbm, bias_hbm, out_hbm, idx_v, rows_v, out_v, bias_v, sem):
    wid = lax.axis_index("c") * NS + lax.axis_index("s")

    for f in range(NUM_FIELDS):
        pltpu.make_async_copy(
            xt_hbm.at[pl.ds(f * BATCH + wid * B_PER_W, B_PER_W)],
            idx_v.at[pl.ds(f * B_PER_W, B_PER_W)],
            sem,
        ).start()
    pltpu.sync_copy(bias_hbm, bias_v)
    for f in range(NUM_FIELDS):
        pltpu.make_async_copy(
            xt_hbm.at[pl.ds(f * BATCH + wid * B_PER_W, B_PER_W)],
            idx_v.at[pl.ds(f * B_PER_W, B_PER_W)],
            sem,
        ).wait()

    def add_offsets(f, carry):
        off = f * FIELD_DIM

        def inner(c, carry2):
            o = f * B_PER_W + c * L
            idx_v[pl.ds(o, L)] = idx_v[pl.ds(o, L)] + off
            return carry2

        return lax.fori_loop(0, B_PER_W // L, inner, carry)

    lax.fori_loop(0, NUM_FIELDS, add_offsets, 0)

    def fire(j, carry):
        o = j * CHUNK
        pltpu.make_async_copy(
            flat_hbm.at[idx_v.at[pl.ds(o, CHUNK)]],
            rows_v.at[pl.ds(o, CHUNK)],
            sem,
        ).start()
        return carry

    lax.fori_loop(0, N_CHUNKS, fire, 0)

    def drain(j, carry):
        o = j * CHUNK
        pltpu.make_async_copy(
            flat_hbm.at[idx_v.at[pl.ds(o, CHUNK)]],
            rows_v.at[pl.ds(o, CHUNK)],
            sem,
        ).wait()
        return carry

    lax.fori_loop(0, N_CHUNKS, drain, 0)

    def reduce(c, carry):
        o = c * L
        acc = bias_v[...]
        for f in range(NUM_FIELDS):
            acc = acc + rows_v[pl.ds(f * B_PER_W + o, L)]
        out_v[pl.ds(o, L)] = acc
        return carry

    lax.fori_loop(0, B_PER_W // L, reduce, 0)

    pltpu.sync_copy(out_v, out_hbm.at[pl.ds(wid * B_PER_W, B_PER_W)])


@jax.jit
def _run(xt_flat, tab_flat, bias16):
    mesh = plsc.VectorSubcoreMesh(core_axis_name="c", subcore_axis_name="s")
    return pl.kernel(
        _gather_body,
        out_type=jax.ShapeDtypeStruct((BATCH,), jnp.float32),
        mesh=mesh,
        scratch_types=[
            pltpu.VMEM((N_PER_W,), jnp.int32),
            pltpu.VMEM((N_PER_W,), jnp.float32),
            pltpu.VMEM((B_PER_W,), jnp.float32),
            pltpu.VMEM((L,), jnp.float32),
            pltpu.SemaphoreType.DMA,
        ],
    )(xt_flat, tab_flat, bias16)


def kernel(x, table, bias):
    xt_flat = x.T.reshape(-1)
    tab_flat = table.reshape(-1)
    bias16 = jnp.broadcast_to(bias, (L,))
    out = _run(xt_flat, tab_flat, bias16)
    return out.reshape(BATCH, 1)


# speedup vs baseline: 1.1456x; 1.0228x over previous
import jax
import jax.numpy as jnp
from jax import lax
from jax.experimental import pallas as pl
from jax.experimental.pallas import tpu as pltpu
from jax.experimental.pallas import tpu_sc as plsc

NUM_FIELDS = 26
FIELD_DIM = 100000
TOTAL_ROWS = NUM_FIELDS * FIELD_DIM
BATCH = 16384
L = 16
NC, NS = 2, 16
NW = NC * NS
B_PER_W = BATCH // NW
N_PER_W = B_PER_W * NUM_FIELDS
CHUNK = 128
N_CHUNKS = N_PER_W // CHUNK
UNROLL = 4


def _index_body(xt_hbm, idx_hbm, idx_v, sem):
    wid = lax.axis_index("c") * NS + lax.axis_index("s")

    for f in range(NUM_FIELDS):
        pltpu.make_async_copy(
            xt_hbm.at[pl.ds(f * BATCH + wid * B_PER_W, B_PER_W)],
            idx_v.at[pl.ds(f * B_PER_W, B_PER_W)],
            sem,
        ).start()
    for f in range(NUM_FIELDS):
        pltpu.make_async_copy(
            xt_hbm.at[pl.ds(f * BATCH + wid * B_PER_W, B_PER_W)],
            idx_v.at[pl.ds(f * B_PER_W, B_PER_W)],
            sem,
        ).wait()

    def add_offsets(f, carry):
        off = f * FIELD_DIM

        def inner(c, carry2):
            for u in range(UNROLL):
                o = f * B_PER_W + (c * UNROLL + u) * L
                idx_v[pl.ds(o, L)] = idx_v[pl.ds(o, L)] + off
            return carry2

        return lax.fori_loop(0, B_PER_W // (L * UNROLL), inner, carry)

    lax.fori_loop(0, NUM_FIELDS, add_offsets, 0)

    pltpu.sync_copy(idx_v, idx_hbm.at[pl.ds(wid * N_PER_W, N_PER_W)])


def _gather_body(idx_hbm, tab_hbm, bias_hbm, out_hbm,
                 idx_v, rows_v, out_v, bias_v, sem):
    wid = lax.axis_index("c") * NS + lax.axis_index("s")

    pltpu.sync_copy(idx_hbm.at[pl.ds(wid * N_PER_W, N_PER_W)], idx_v)
    pltpu.sync_copy(bias_hbm, bias_v)

    def fire(j, carry):
        o = j * CHUNK
        pltpu.make_async_copy(
            tab_hbm.at[idx_v.at[pl.ds(o, CHUNK)]],
            rows_v.at[pl.ds(o, CHUNK)],
            sem,
        ).start()
        return carry

    lax.fori_loop(0, N_CHUNKS, fire, 0)

    def drain(j, carry):
        o = j * CHUNK
        pltpu.make_async_copy(
            tab_hbm.at[idx_v.at[pl.ds(o, CHUNK)]],
            rows_v.at[pl.ds(o, CHUNK)],
            sem,
        ).wait()
        return carry

    lax.fori_loop(0, N_CHUNKS, drain, 0)

    def reduce(c, carry):
        o = c * L
        acc = bias_v[...]
        for f in range(NUM_FIELDS):
            acc = acc + rows_v[pl.ds(f * B_PER_W + o, L)]
        out_v[pl.ds(o, L)] = acc
        return carry

    lax.fori_loop(0, B_PER_W // L, reduce, 0)

    pltpu.sync_copy(out_v, out_hbm.at[pl.ds(wid * B_PER_W, B_PER_W)])


@jax.jit
def _run(xt_flat, tab_flat, bias16):
    mesh = plsc.VectorSubcoreMesh(core_axis_name="c", subcore_axis_name="s")
    idx = pl.kernel(
        _index_body,
        out_type=jax.ShapeDtypeStruct((BATCH * NUM_FIELDS,), jnp.int32),
        mesh=mesh,
        scratch_types=[
            pltpu.VMEM((N_PER_W,), jnp.int32),
            pltpu.SemaphoreType.DMA,
        ],
    )(xt_flat)
    return pl.kernel(
        _gather_body,
        out_type=jax.ShapeDtypeStruct((BATCH,), jnp.float32),
        mesh=mesh,
        scratch_types=[
            pltpu.VMEM((N_PER_W,), jnp.int32),
            pltpu.VMEM((N_PER_W,), jnp.float32),
            pltpu.VMEM((B_PER_W,), jnp.float32),
            pltpu.VMEM((L,), jnp.float32),
            pltpu.SemaphoreType.DMA,
        ],
    )(idx, tab_flat, bias16)


def kernel(x, table, bias):
    xt_flat = x.T.reshape(-1)
    tab_flat = table.reshape(-1)
    bias16 = jnp.broadcast_to(bias, (L,))
    out = _run(xt_flat, tab_flat, bias16)
    return out.reshape(BATCH, 1)


# speedup vs baseline: 2.2316x; 1.9479x over previous
import jax
import jax.numpy as jnp
from jax import lax
from jax.experimental import pallas as pl
from jax.experimental.pallas import tpu as pltpu
from jax.experimental.pallas import tpu_sc as plsc

NUM_FIELDS = 26
FIELD_DIM = 100000
TOTAL_ROWS = NUM_FIELDS * FIELD_DIM
BATCH = 16384
L = 16
NC, NS = 2, 16
NW = NC * NS
B_PER_W = BATCH // NW
N_PER_W = B_PER_W * NUM_FIELDS
CHUNK = 128
N_CHUNKS = N_PER_W // CHUNK
UNROLL = 4
GROUP_FIELDS = 2
N_GROUPS = NUM_FIELDS // GROUP_FIELDS
N_PER_G = GROUP_FIELDS * B_PER_W
CHUNKS_PER_G = N_PER_G // CHUNK


def _index_body(xt_hbm, idx_hbm, idx_v, sem):
    wid = lax.axis_index("c") * NS + lax.axis_index("s")

    for f in range(NUM_FIELDS):
        pltpu.make_async_copy(
            xt_hbm.at[pl.ds(f * BATCH + wid * B_PER_W, B_PER_W)],
            idx_v.at[pl.ds(f * B_PER_W, B_PER_W)],
            sem,
        ).start()
    for f in range(NUM_FIELDS):
        pltpu.make_async_copy(
            xt_hbm.at[pl.ds(f * BATCH + wid * B_PER_W, B_PER_W)],
            idx_v.at[pl.ds(f * B_PER_W, B_PER_W)],
            sem,
        ).wait()

    def add_offsets(f, carry):
        off = (f % GROUP_FIELDS) * FIELD_DIM

        def inner(c, carry2):
            for u in range(UNROLL):
                o = f * B_PER_W + (c * UNROLL + u) * L
                idx_v[pl.ds(o, L)] = idx_v[pl.ds(o, L)] + off
            return carry2

        return lax.fori_loop(0, B_PER_W // (L * UNROLL), inner, carry)

    lax.fori_loop(0, NUM_FIELDS, add_offsets, 0)

    pltpu.sync_copy(idx_v, idx_hbm.at[pl.ds(wid * N_PER_W, N_PER_W)])


def _gather_body(idx_hbm, *refs):
    tabs = refs[:N_GROUPS]
    bias_hbm, out_hbm, idx_v, rows_v, out_v, bias_v, sem = refs[N_GROUPS:]
    wid = lax.axis_index("c") * NS + lax.axis_index("s")

    pltpu.sync_copy(idx_hbm.at[pl.ds(wid * N_PER_W, N_PER_W)], idx_v)
    pltpu.sync_copy(bias_hbm, bias_v)

    for g in range(N_GROUPS):
        def fire(j, carry, g=g):
            o = g * N_PER_G + j * CHUNK
            pltpu.make_async_copy(
                tabs[g].at[idx_v.at[pl.ds(o, CHUNK)]],
                rows_v.at[pl.ds(o, CHUNK)],
                sem,
            ).start()
            return carry

        lax.fori_loop(0, CHUNKS_PER_G, fire, 0)

    for g in range(N_GROUPS):
        def drain(j, carry, g=g):
            o = g * N_PER_G + j * CHUNK
            pltpu.make_async_copy(
                tabs[g].at[idx_v.at[pl.ds(o, CHUNK)]],
                rows_v.at[pl.ds(o, CHUNK)],
                sem,
            ).wait()
            return carry

        lax.fori_loop(0, CHUNKS_PER_G, drain, 0)

    def reduce(c, carry):
        o = c * L
        acc = bias_v[...]
        for f in range(NUM_FIELDS):
            acc = acc + rows_v[pl.ds(f * B_PER_W + o, L)]
        out_v[pl.ds(o, L)] = acc
        return carry

    lax.fori_loop(0, B_PER_W // L, reduce, 0)

    pltpu.sync_copy(out_v, out_hbm.at[pl.ds(wid * B_PER_W, B_PER_W)])


@jax.jit
def _run(xt_flat, tabs, bias16):
    mesh = plsc.VectorSubcoreMesh(core_axis_name="c", subcore_axis_name="s")
    idx = pl.kernel(
        _index_body,
        out_type=jax.ShapeDtypeStruct((BATCH * NUM_FIELDS,), jnp.int32),
        mesh=mesh,
        scratch_types=[
            pltpu.VMEM((N_PER_W,), jnp.int32),
            pltpu.SemaphoreType.DMA,
        ],
    )(xt_flat)
    return pl.kernel(
        _gather_body,
        out_type=jax.ShapeDtypeStruct((BATCH,), jnp.float32),
        mesh=mesh,
        scratch_types=[
            pltpu.VMEM((N_PER_W,), jnp.int32),
            pltpu.VMEM((N_PER_W,), jnp.float32),
            pltpu.VMEM((B_PER_W,), jnp.float32),
            pltpu.VMEM((L,), jnp.float32),
            pltpu.SemaphoreType.DMA,
        ],
    )(idx, *tabs, bias16)


def kernel(x, table, bias):
    xt_flat = x.T.reshape(-1)
    tabs = tuple(
        table[g * GROUP_FIELDS * FIELD_DIM:(g + 1) * GROUP_FIELDS * FIELD_DIM].reshape(-1)
        for g in range(N_GROUPS)
    )
    bias16 = jnp.broadcast_to(bias, (L,))
    out = _run(xt_flat, tabs, bias16)
    return out.reshape(BATCH, 1)
